# Initial kernel scaffold; baseline (speedup 1.0000x reference)
import jax
import jax.numpy as jnp
from jax.experimental import pallas as pl


def kernel(pos, rgb, normals, batch, ptr, params):
    raise NotImplementedError("write your pallas kernel here")


# speedup vs baseline: 2.6874x; 2.6874x over previous
import functools

import jax
import jax.numpy as jnp
from jax import lax
from jax.experimental import pallas as pl
from jax.experimental.pallas import tpu as pltpu
from jax.experimental.pallas import tpu_sc as plsc

N0 = 16384
N1 = 4096
N2 = 1024
HID = 64
K_NBR = 16
QB = 128

_F32 = jnp.float32
_INF = float('inf')
_IMAX = 2**31 - 1


def _dot(a, b):
    return jnp.dot(a, b, preferred_element_type=jnp.float32)


def _encode_body(rgb, nrm, pos, wce, bce, wne, bne, we_r, we_n, we_p, be,
                 wm0x, wm0p, x_out, y0_out):
    def mesh_enc(tok, W, b):
        acc = None
        for j in range(4):
            h = jax.nn.relu(_dot(tok[:, 3 * j:3 * j + 3], W[...]) + b[...])
            acc = h if acc is None else jnp.maximum(acc, h)
        return acc

    xr = mesh_enc(rgb[...], wce, bce)
    xn = mesh_enc(nrm[...], wne, bne)
    p = pos[...]
    x = jax.nn.relu(_dot(xr, we_r[...]) + _dot(xn, we_n[...]) +
                    _dot(p, we_p[...]) + be[...])
    x_out[...] = x
    y0 = _dot(x, wm0x[...]) + _dot(p, wm0p[...])
    y0_out[...] = jnp.concatenate([y0, jnp.zeros_like(y0)], axis=1)


def _encode(rgb, normals, pos, p):
    RB = 2048
    grid = N0 // RB
    half = HID // 2
    we_r = p['W_emb'][:half]
    we_n = p['W_emb'][half:2 * half]
    we_p = p['W_emb'][2 * half:]
    wm0x = p['W_msg0'][:HID]
    wm0p = p['W_msg0'][HID:]
    row = lambda i: (i, 0)
    full = lambda i: (0, 0)
    out = pl.pallas_call(
        _encode_body,
        grid=(grid,),
        in_specs=[
            pl.BlockSpec((RB, 12), row),
            pl.BlockSpec((RB, 12), row),
            pl.BlockSpec((RB, 3), row),
            pl.BlockSpec((3, half), full),
            pl.BlockSpec((1, half), full),
            pl.BlockSpec((3, half), full),
            pl.BlockSpec((1, half), full),
            pl.BlockSpec((half, HID), full),
            pl.BlockSpec((half, HID), full),
            pl.BlockSpec((3, HID), full),
            pl.BlockSpec((1, HID), full),
            pl.BlockSpec((HID, HID), full),
            pl.BlockSpec((3, HID), full),
        ],
        out_specs=[
            pl.BlockSpec((RB, HID), row),
            pl.BlockSpec((RB, 2 * HID), row),
        ],
        out_shape=[
            jax.ShapeDtypeStruct((N0, HID), _F32),
            jax.ShapeDtypeStruct((N0, 2 * HID), _F32),
        ],
    )(rgb, normals, pos,
      p['W_ce'], p['b_ce'][None, :], p['W_ne'], p['b_ne'][None, :],
      we_r, we_n, we_p, p['b_emb'][None, :], wm0x, wm0p)
    return out


def _knn_body(qpt, bpos, idx_out, d2_s, *, nb, k, ch):
    q = qpt[...]
    b = bpos[...]
    q2 = jnp.sum(q * q, axis=0, keepdims=True)
    b2 = jnp.sum(b * b, axis=1, keepdims=True)
    d2_s[...] = b2 + q2 - 2.0 * _dot(b, q)

    nch = nb // ch
    lv = jnp.full((1, QB), -_INF, _F32)
    li = jnp.full((1, QB), -1, jnp.int32)
    for t in range(k):
        def body(c, carry):
            bv, bi = carry
            d = d2_s[pl.ds(c * ch, ch), :]
            ridx = lax.broadcasted_iota(jnp.int32, (ch, QB), 0) + c * ch
            elig = (d > lv) | ((d == lv) & (ridx > li))
            cand = jnp.where(elig, d, _INF)
            cmin = jnp.min(cand, axis=0, keepdims=True)
            cidx = jnp.min(jnp.where(cand == cmin, ridx, _IMAX),
                           axis=0, keepdims=True)
            take = (cmin < bv) | ((cmin == bv) & (cidx < bi))
            return jnp.where(take, cmin, bv), jnp.where(take, cidx, bi)

        bv, bi = lax.fori_loop(0, nch, body,
                               (jnp.full((1, QB), _INF, _F32),
                                jnp.full((1, QB), _IMAX, jnp.int32)))
        idx_out[pl.ds(t, 1), :] = bi
        lv, li = bv, bi


def _knn_topk(qpos, bpos, k):
    nq = qpos.shape[0]
    nb = bpos.shape[0]
    ch = 256
    grid = nq // QB
    body = functools.partial(_knn_body, nb=nb, k=k, ch=ch)
    idx = pl.pallas_call(
        body,
        grid=(grid,),
        in_specs=[
            pl.BlockSpec((3, QB), lambda i: (0, i)),
            pl.BlockSpec((nb, 3), lambda i: (0, 0)),
        ],
        out_specs=pl.BlockSpec((k, QB), lambda i: (0, i)),
        out_shape=jax.ShapeDtypeStruct((k, nq), jnp.int32),
        scratch_shapes=[pltpu.VMEM((nb, QB), _F32)],
    )(qpos.T, bpos)
    return idx


def _sc_gather(table, idx):
    V, D = table.shape
    B = idx.shape[0]
    NC, NS = 2, 16
    NW = NC * NS
    GCH = 128
    assert B % (NW * GCH) == 0 and D % 128 == 0
    nchunk = B // (NW * GCH)
    idx2 = idx.reshape(-1, GCH)
    mesh = plsc.VectorSubcoreMesh(core_axis_name="c", subcore_axis_name="s",
                                  num_cores=NC, num_subcores=NS)

    @functools.partial(
        pl.kernel, mesh=mesh,
        out_type=jax.ShapeDtypeStruct((B, D), _F32),
        scratch_types=[
            pltpu.VMEM((nchunk, GCH), jnp.int32),
            pltpu.VMEM((GCH, D), _F32),
            pltpu.VMEM((GCH, D), _F32),
            pltpu.SemaphoreType.DMA,
            pltpu.SemaphoreType.DMA,
            pltpu.SemaphoreType.DMA,
            pltpu.SemaphoreType.DMA,
        ],
    )
    def gather_k(table_hbm, idx_hbm, out_hbm, idx_v, buf0, buf1,
                 sg0, sg1, sw0, sw1):
        wid = lax.axis_index("s") * NC + lax.axis_index("c")
        rbase = wid * nchunk
        pltpu.sync_copy(idx_hbm.at[pl.ds(rbase, nchunk)], idx_v)
        bufs = (buf0, buf1)
        gsems = (sg0, sg1)
        wsems = (sw0, sw1)
        gdesc = [None, None]
        wdesc = [None, None]
        for j in range(nchunk):
            s = j % 2
            if wdesc[s] is not None:
                wdesc[s].wait()
            gdesc[s] = pltpu.async_copy(table_hbm.at[idx_v.at[j]],
                                        bufs[s], gsems[s])
            if j > 0:
                sp = (j - 1) % 2
                gdesc[sp].wait()
                wdesc[sp] = pltpu.async_copy(
                    bufs[sp], out_hbm.at[pl.ds((rbase + j - 1) * GCH, GCH)],
                    wsems[sp])
        sl = (nchunk - 1) % 2
        gdesc[sl].wait()
        wdesc[sl] = pltpu.async_copy(
            bufs[sl], out_hbm.at[pl.ds((rbase + nchunk - 1) * GCH, GCH)],
            wsems[sl])
        for d in wdesc:
            if d is not None:
                d.wait()

    return gather_k(table, idx2)


def _gapl_body(g, qpos, xp, wmp, bm, a, ws, bs, wr, br, wres, bres,
               *args, with_y):
    if with_y:
        wyx, wyp = args[0], args[1]
        xn_out, y_out = args[2], args[3]
    else:
        xn_out, = args
    z = _dot(qpos[...], wmp[...])
    bmv = bm[...]
    av = a[...]
    c = z.shape[1]
    msgs = []
    logits = []
    for kk in range(K_NBR):
        m = jax.nn.relu(g[kk][:, :c] - z + bmv)
        msgs.append(m)
        logits.append(jnp.sum(m * av, axis=1, keepdims=True))
    lg = jnp.concatenate(logits, axis=1)
    mx = jnp.max(lg, axis=1, keepdims=True)
    e = jnp.exp(lg - mx)
    denom = jnp.sum(e, axis=1, keepdims=True)
    agg = None
    for kk in range(K_NBR):
        contrib = e[:, kk:kk + 1] * msgs[kk]
        agg = contrib if agg is None else agg + contrib
    agg = agg / denom
    xh = jax.nn.relu(agg + _dot(xp[...], ws[...]) + bs[...])
    xn = jax.nn.relu(_dot(xh, wr[...]) + br[...])
    xn = xn + jax.nn.relu(_dot(xn, wres[...]) + bres[...])
    xn_out[...] = xn
    if with_y:
        y_out[...] = _dot(xn, wyx[...]) + _dot(qpos[...], wyp[...])


def _gapl(g3, qpos, xpool, wm_p, bm, a, ws, bs, wr, br, wres, bres,
          wyx=None, wyp=None):
    nq, c = xpool.shape
    c2 = 2 * c
    cg = g3.shape[2]
    grid = nq // QB
    with_y = wyx is not None
    row = lambda i: (i, 0)
    full = lambda i: (0, 0)
    in_specs = [
        pl.BlockSpec((K_NBR, QB, cg), lambda i: (0, i, 0)),
        pl.BlockSpec((QB, 3), row),
        pl.BlockSpec((QB, c), row),
        pl.BlockSpec((3, c), full),
        pl.BlockSpec((1, c), full),
        pl.BlockSpec((1, c), full),
        pl.BlockSpec((c, c), full),
        pl.BlockSpec((1, c), full),
        pl.BlockSpec((c, c2), full),
        pl.BlockSpec((1, c2), full),
        pl.BlockSpec((c2, c2), full),
        pl.BlockSpec((1, c2), full),
    ]
    args = [g3, qpos, xpool, wm_p, bm[None, :], a[None, :], ws, bs[None, :],
            wr, br[None, :], wres, bres[None, :]]
    out_specs = [pl.BlockSpec((QB, c2), row)]
    out_shape = [jax.ShapeDtypeStruct((nq, c2), _F32)]
    if with_y:
        in_specs += [pl.BlockSpec((c2, c2), full), pl.BlockSpec((3, c2), full)]
        args += [wyx, wyp]
        out_specs.append(pl.BlockSpec((QB, c2), row))
        out_shape.append(jax.ShapeDtypeStruct((nq, c2), _F32))
    body = functools.partial(_gapl_body, with_y=with_y)
    return pl.pallas_call(
        body,
        grid=(grid,),
        in_specs=in_specs,
        out_specs=out_specs,
        out_shape=out_shape,
    )(*args)


def _dec_body(qpos, bpt, xb, xskip, wda, wdb, bd, wdr, bdr,
              *args, with_head):
    if with_head:
        wm1, bm1, wm2, bm2, wm3, bm3, out = args
    else:
        out, = args
    qp = qpos[...]
    bp = bpt[...]
    q2 = jnp.sum(qp * qp, axis=1, keepdims=True)
    b2 = jnp.sum(bp * bp, axis=0, keepdims=True)
    d2 = q2 + b2 - 2.0 * _dot(qp, bp)
    m1 = jnp.min(d2, axis=1, keepdims=True)
    d2a = jnp.where(d2 <= m1, _INF, d2)
    m2 = jnp.min(d2a, axis=1, keepdims=True)
    d2b = jnp.where(d2a <= m2, _INF, d2a)
    m3 = jnp.min(d2b, axis=1, keepdims=True)
    w = jnp.where(d2 <= m3, 1.0 / jnp.maximum(d2, 1e-16), 0.0)
    up = _dot(w, xb[...]) / jnp.sum(w, axis=1, keepdims=True)
    xi = jax.nn.relu(_dot(xskip[...], wda[...]) + _dot(up, wdb[...]) + bd[...])
    xi = xi + jax.nn.relu(_dot(xi, wdr[...]) + bdr[...])
    if with_head:
        h = jax.nn.relu(_dot(xi, wm1[...]) + bm1[...])
        h = jax.nn.relu(_dot(h, wm2[...]) + bm2[...])
        out[...] = _dot(h, wm3[...]) + bm3[...]
    else:
        out[...] = xi


def _decode(qpos, bpos, xb, xskip, wd, bd, wdr, bdr, head=None):
    nq = qpos.shape[0]
    nb, cb = xb.shape
    cs = xskip.shape[1]
    co = wd.shape[1]
    grid = nq // QB
    wda = wd[:cs]
    wdb = wd[cs:]
    row = lambda i: (i, 0)
    full = lambda i: (0, 0)
    in_specs = [
        pl.BlockSpec((QB, 3), row),
        pl.BlockSpec((3, nb), full),
        pl.BlockSpec((nb, cb), full),
        pl.BlockSpec((QB, cs), row),
        pl.BlockSpec((cs, co), full),
        pl.BlockSpec((cb, co), full),
        pl.BlockSpec((1, co), full),
        pl.BlockSpec((co, co), full),
        pl.BlockSpec((1, co), full),
    ]
    args = [qpos, bpos.T, xb, xskip, wda, wdb, bd[None, :], wdr, bdr[None, :]]
    if head is not None:
        wm1, bm1, wm2, bm2, wm3, bm3 = head
        cm = wm1.shape[1]
        cl = wm3.shape[1]
        in_specs += [
            pl.BlockSpec((co, cm), full), pl.BlockSpec((1, cm), full),
            pl.BlockSpec((cm, cm), full), pl.BlockSpec((1, cm), full),
            pl.BlockSpec((cm, cl), full), pl.BlockSpec((1, cl), full),
        ]
        args += [wm1, bm1[None, :], wm2, bm2[None, :], wm3, bm3[None, :]]
        out_c = cl
    else:
        out_c = co
    body = functools.partial(_dec_body, with_head=head is not None)
    return pl.pallas_call(
        body,
        grid=(grid,),
        in_specs=in_specs,
        out_specs=pl.BlockSpec((QB, out_c), row),
        out_shape=jax.ShapeDtypeStruct((nq, out_c), _F32),
    )(*args)


def kernel(pos, rgb, normals, batch, ptr, params):
    p = params
    x, y0 = _encode(rgb, normals, pos, p)

    pos1 = pos.reshape(N1, 4, 3)[:, 0, :]
    pos2 = pos1.reshape(N2, 4, 3)[:, 0, :]

    idx0 = _knn_topk(pos1, pos, K_NBR)
    g0 = _sc_gather(y0, idx0.reshape(-1))
    g0 = g0.reshape(K_NBR, N1, 2 * HID)
    xpool0 = x.reshape(N1, 4, HID)[:, 0, :]
    wm1x = p['W_msg1'][:2 * HID]
    wm1p = p['W_msg1'][2 * HID:]
    x1, y1 = _gapl(g0, pos1, xpool0, p['W_msg0'][HID:], p['b_msg0'],
                   p['a0'], p['W_self0'], p['b_self0'],
                   p['W_r0'], p['b_r0'], p['W_res0'], p['b_res0'],
                   wyx=wm1x, wyp=wm1p)

    idx1 = _knn_topk(pos2, pos1, K_NBR)
    g1 = _sc_gather(y1, idx1.reshape(-1))
    g1 = g1.reshape(K_NBR, N2, 2 * HID)
    xpool1 = x1.reshape(N2, 4, 2 * HID)[:, 0, :]
    (x2,) = _gapl(g1, pos2, xpool1, wm1p, p['b_msg1'],
                  p['a1'], p['W_self1'], p['b_self1'],
                  p['W_r1'], p['b_r1'], p['W_res1'], p['b_res1'])

    xo1 = _decode(pos1, pos2, x2, x1, p['W_d0'], p['b_d0'],
                  p['W_dr0'], p['b_dr0'])
    wm3 = jnp.pad(p['W_m3'], ((0, 0), (0, 128 - p['W_m3'].shape[1])))
    bm3 = jnp.pad(p['b_m3'], (0, 128 - p['b_m3'].shape[0]))
    out = _decode(pos, pos1, xo1, x, p['W_d1'], p['b_d1'],
                  p['W_dr1'], p['b_dr1'],
                  head=(p['W_m1'], p['b_m1'], p['W_m2'], p['b_m2'], wm3, bm3))
    return out[:, :p['W_m3'].shape[1]]


# speedup vs baseline: 3.7991x; 1.4136x over previous
import functools

import jax
import jax.numpy as jnp
from jax import lax
from jax.experimental import pallas as pl
from jax.experimental.pallas import tpu as pltpu
from jax.experimental.pallas import tpu_sc as plsc

N0 = 16384
N1 = 4096
N2 = 1024
HID = 64
K_NBR = 16
QB = 128

_F32 = jnp.float32
_INF = float('inf')
_IMAX = 2**31 - 1


def _dot(a, b):
    return jnp.dot(a, b, preferred_element_type=jnp.float32)


def _encode_body(rgb, nrm, pos, wce, bce, wne, bne, we_r, we_n, we_p, be,
                 wm0x, wm0p, x_out, y0_out):
    def mesh_enc(tok, W, b):
        acc = None
        for j in range(4):
            h = jax.nn.relu(_dot(tok[:, 3 * j:3 * j + 3], W[...]) + b[...])
            acc = h if acc is None else jnp.maximum(acc, h)
        return acc

    xr = mesh_enc(rgb[...], wce, bce)
    xn = mesh_enc(nrm[...], wne, bne)
    p = pos[...]
    x = jax.nn.relu(_dot(xr, we_r[...]) + _dot(xn, we_n[...]) +
                    _dot(p, we_p[...]) + be[...])
    x_out[...] = x
    y0 = _dot(x, wm0x[...]) + _dot(p, wm0p[...])
    y0_out[...] = jnp.concatenate([y0, jnp.zeros_like(y0)], axis=1)


def _encode(rgb, normals, pos, p):
    RB = 2048
    grid = N0 // RB
    half = HID // 2
    we_r = p['W_emb'][:half]
    we_n = p['W_emb'][half:2 * half]
    we_p = p['W_emb'][2 * half:]
    wm0x = p['W_msg0'][:HID]
    wm0p = p['W_msg0'][HID:]
    row = lambda i: (i, 0)
    full = lambda i: (0, 0)
    out = pl.pallas_call(
        _encode_body,
        grid=(grid,),
        in_specs=[
            pl.BlockSpec((RB, 12), row),
            pl.BlockSpec((RB, 12), row),
            pl.BlockSpec((RB, 3), row),
            pl.BlockSpec((3, half), full),
            pl.BlockSpec((1, half), full),
            pl.BlockSpec((3, half), full),
            pl.BlockSpec((1, half), full),
            pl.BlockSpec((half, HID), full),
            pl.BlockSpec((half, HID), full),
            pl.BlockSpec((3, HID), full),
            pl.BlockSpec((1, HID), full),
            pl.BlockSpec((HID, HID), full),
            pl.BlockSpec((3, HID), full),
        ],
        out_specs=[
            pl.BlockSpec((RB, HID), row),
            pl.BlockSpec((RB, 2 * HID), row),
        ],
        out_shape=[
            jax.ShapeDtypeStruct((N0, HID), _F32),
            jax.ShapeDtypeStruct((N0, 2 * HID), _F32),
        ],
    )(rgb, normals, pos,
      p['W_ce'], p['b_ce'][None, :], p['W_ne'], p['b_ne'][None, :],
      we_r, we_n, we_p, p['b_emb'][None, :], wm0x, wm0p)
    return out


def _knn_body(qpt, bpos, idx_out, d2_s, *, nb, k, ch):
    q = qpt[...]
    b = bpos[...]
    b2 = jnp.sum(b * b, axis=1, keepdims=True)
    d2_s[...] = b2 - 2.0 * _dot(b, q)

    nch = nb // ch
    lv = jnp.full((1, QB), -_INF, _F32)
    for t in range(k // 2):
        def body(c, carry):
            bv1, bi1, bv2, bi2 = carry
            d = d2_s[pl.ds(c * ch, ch), :]
            ridx = lax.broadcasted_iota(jnp.int32, (ch, QB), 0) + c * ch
            cand = jnp.where(d > lv, d, _INF)
            m1 = jnp.min(cand, axis=0, keepdims=True)
            i1 = jnp.min(jnp.where(cand == m1, ridx, _IMAX),
                         axis=0, keepdims=True)
            cand2 = jnp.where((cand == m1) & (ridx == i1), _INF, cand)
            m2 = jnp.min(cand2, axis=0, keepdims=True)
            i2 = jnp.min(jnp.where(cand2 == m2, ridx, _IMAX),
                         axis=0, keepdims=True)
            tk = m1 < bv1
            fv = jnp.where(tk, m1, bv1)
            fi = jnp.where(tk, i1, bi1)
            sav = jnp.where(tk, bv1, m1)
            sai = jnp.where(tk, bi1, i1)
            sbv = jnp.where(tk, m2, bv2)
            sbi = jnp.where(tk, i2, bi2)
            u = sbv < sav
            return (fv, fi, jnp.where(u, sbv, sav), jnp.where(u, sbi, sai))

        bv1, bi1, bv2, bi2 = lax.fori_loop(
            0, nch, body,
            (jnp.full((1, QB), _INF, _F32),
             jnp.full((1, QB), _IMAX, jnp.int32),
             jnp.full((1, QB), _INF, _F32),
             jnp.full((1, QB), _IMAX, jnp.int32)))
        idx_out[pl.ds(2 * t, 1), :] = bi1
        idx_out[pl.ds(2 * t + 1, 1), :] = bi2
        lv = bv2


def _knn_topk(qpos, bpos, k):
    nq = qpos.shape[0]
    nb = bpos.shape[0]
    ch = 256
    grid = nq // QB
    body = functools.partial(_knn_body, nb=nb, k=k, ch=ch)
    idx = pl.pallas_call(
        body,
        grid=(grid,),
        in_specs=[
            pl.BlockSpec((3, QB), lambda i: (0, i)),
            pl.BlockSpec((nb, 3), lambda i: (0, 0)),
        ],
        out_specs=pl.BlockSpec((k, QB), lambda i: (0, i)),
        out_shape=jax.ShapeDtypeStruct((k, nq), jnp.int32),
        scratch_shapes=[pltpu.VMEM((nb, QB), _F32)],
    )(qpos.T, bpos)
    return idx


def _sc_gather(table, idx):
    V, D = table.shape
    B = idx.shape[0]
    NC, NS = 2, 16
    NW = NC * NS
    GCH = 128
    assert B % (NW * GCH) == 0 and D % 128 == 0
    nchunk = B // (NW * GCH)
    idx2 = idx.reshape(-1, GCH)
    mesh = plsc.VectorSubcoreMesh(core_axis_name="c", subcore_axis_name="s",
                                  num_cores=NC, num_subcores=NS)

    @functools.partial(
        pl.kernel, mesh=mesh,
        out_type=jax.ShapeDtypeStruct((B, D), _F32),
        scratch_types=[
            pltpu.VMEM((nchunk, GCH), jnp.int32),
            pltpu.VMEM((GCH, D), _F32),
            pltpu.VMEM((GCH, D), _F32),
            pltpu.SemaphoreType.DMA,
            pltpu.SemaphoreType.DMA,
            pltpu.SemaphoreType.DMA,
            pltpu.SemaphoreType.DMA,
        ],
    )
    def gather_k(table_hbm, idx_hbm, out_hbm, idx_v, buf0, buf1,
                 sg0, sg1, sw0, sw1):
        wid = lax.axis_index("s") * NC + lax.axis_index("c")
        rbase = wid * nchunk
        pltpu.sync_copy(idx_hbm.at[pl.ds(rbase, nchunk)], idx_v)
        bufs = (buf0, buf1)
        gsems = (sg0, sg1)
        wsems = (sw0, sw1)
        gdesc = [None, None]
        wdesc = [None, None]
        for j in range(nchunk):
            s = j % 2
            if wdesc[s] is not None:
                wdesc[s].wait()
            gdesc[s] = pltpu.async_copy(table_hbm.at[idx_v.at[j]],
                                        bufs[s], gsems[s])
            if j > 0:
                sp = (j - 1) % 2
                gdesc[sp].wait()
                wdesc[sp] = pltpu.async_copy(
                    bufs[sp], out_hbm.at[pl.ds((rbase + j - 1) * GCH, GCH)],
                    wsems[sp])
        sl = (nchunk - 1) % 2
        gdesc[sl].wait()
        wdesc[sl] = pltpu.async_copy(
            bufs[sl], out_hbm.at[pl.ds((rbase + nchunk - 1) * GCH, GCH)],
            wsems[sl])
        for d in wdesc:
            if d is not None:
                d.wait()

    return gather_k(table, idx2)


def _gapl_body(g, qpos, xp, wmp, bm, a, ws, bs, wr, br, wres, bres,
               *args, with_y):
    if with_y:
        wyx, wyp = args[0], args[1]
        xn_out, y_out = args[2], args[3]
    else:
        xn_out, = args
    z = _dot(qpos[...], wmp[...])
    bmv = bm[...]
    av = a[...]
    c = z.shape[1]
    msgs = []
    logits = []
    for kk in range(K_NBR):
        m = jax.nn.relu(g[kk][:, :c] - z + bmv)
        msgs.append(m)
        logits.append(jnp.sum(m * av, axis=1, keepdims=True))
    lg = jnp.concatenate(logits, axis=1)
    mx = jnp.max(lg, axis=1, keepdims=True)
    e = jnp.exp(lg - mx)
    denom = jnp.sum(e, axis=1, keepdims=True)
    agg = None
    for kk in range(K_NBR):
        contrib = e[:, kk:kk + 1] * msgs[kk]
        agg = contrib if agg is None else agg + contrib
    agg = agg / denom
    xh = jax.nn.relu(agg + _dot(xp[...], ws[...]) + bs[...])
    xn = jax.nn.relu(_dot(xh, wr[...]) + br[...])
    xn = xn + jax.nn.relu(_dot(xn, wres[...]) + bres[...])
    xn_out[...] = xn
    if with_y:
        y_out[...] = _dot(xn, wyx[...]) + _dot(qpos[...], wyp[...])


def _gapl(g3, qpos, xpool, wm_p, bm, a, ws, bs, wr, br, wres, bres,
          wyx=None, wyp=None):
    nq, c = xpool.shape
    c2 = 2 * c
    cg = g3.shape[2]
    grid = nq // QB
    with_y = wyx is not None
    row = lambda i: (i, 0)
    full = lambda i: (0, 0)
    in_specs = [
        pl.BlockSpec((K_NBR, QB, cg), lambda i: (0, i, 0)),
        pl.BlockSpec((QB, 3), row),
        pl.BlockSpec((QB, c), row),
        pl.BlockSpec((3, c), full),
        pl.BlockSpec((1, c), full),
        pl.BlockSpec((1, c), full),
        pl.BlockSpec((c, c), full),
        pl.BlockSpec((1, c), full),
        pl.BlockSpec((c, c2), full),
        pl.BlockSpec((1, c2), full),
        pl.BlockSpec((c2, c2), full),
        pl.BlockSpec((1, c2), full),
    ]
    args = [g3, qpos, xpool, wm_p, bm[None, :], a[None, :], ws, bs[None, :],
            wr, br[None, :], wres, bres[None, :]]
    out_specs = [pl.BlockSpec((QB, c2), row)]
    out_shape = [jax.ShapeDtypeStruct((nq, c2), _F32)]
    if with_y:
        in_specs += [pl.BlockSpec((c2, c2), full), pl.BlockSpec((3, c2), full)]
        args += [wyx, wyp]
        out_specs.append(pl.BlockSpec((QB, c2), row))
        out_shape.append(jax.ShapeDtypeStruct((nq, c2), _F32))
    body = functools.partial(_gapl_body, with_y=with_y)
    return pl.pallas_call(
        body,
        grid=(grid,),
        in_specs=in_specs,
        out_specs=out_specs,
        out_shape=out_shape,
    )(*args)


def _dec_body(qpos, bpt, xb, xskip, wda, wdb, bd, wdr, bdr,
              *args, with_head):
    if with_head:
        wm1, bm1, wm2, bm2, wm3, bm3, out = args
    else:
        out, = args
    qp = qpos[...]
    bp = bpt[...]
    q2 = jnp.sum(qp * qp, axis=1, keepdims=True)
    b2 = jnp.sum(bp * bp, axis=0, keepdims=True)
    d2 = q2 + b2 - 2.0 * _dot(qp, bp)
    nb = d2.shape[1]
    CHL = 128
    qb = d2.shape[0]
    m1 = jnp.full((qb, CHL), _INF, _F32)
    m2 = jnp.full((qb, CHL), _INF, _F32)
    m3 = jnp.full((qb, CHL), _INF, _F32)
    for j in range(nb // CHL):
        v = d2[:, j * CHL:(j + 1) * CHL]
        t1 = jnp.maximum(m1, v)
        m1 = jnp.minimum(m1, v)
        t2 = jnp.maximum(m2, t1)
        m2 = jnp.minimum(m2, t1)
        m3 = jnp.minimum(m3, t2)
    s = jnp.concatenate([m1, m2, m3], axis=1)
    a1 = jnp.min(s, axis=1, keepdims=True)
    s2 = jnp.where(s <= a1, _INF, s)
    a2 = jnp.min(s2, axis=1, keepdims=True)
    s3 = jnp.where(s2 <= a2, _INF, s2)
    m3g = jnp.min(s3, axis=1, keepdims=True)
    w = jnp.where(d2 <= m3g, 1.0 / jnp.maximum(d2, 1e-16), 0.0)
    up = _dot(w, xb[...]) / jnp.sum(w, axis=1, keepdims=True)
    xi = jax.nn.relu(_dot(xskip[...], wda[...]) + _dot(up, wdb[...]) + bd[...])
    xi = xi + jax.nn.relu(_dot(xi, wdr[...]) + bdr[...])
    if with_head:
        h = jax.nn.relu(_dot(xi, wm1[...]) + bm1[...])
        h = jax.nn.relu(_dot(h, wm2[...]) + bm2[...])
        out[...] = _dot(h, wm3[...]) + bm3[...]
    else:
        out[...] = xi


def _decode(qpos, bpos, xb, xskip, wd, bd, wdr, bdr, head=None):
    nq = qpos.shape[0]
    nb, cb = xb.shape
    cs = xskip.shape[1]
    co = wd.shape[1]
    grid = nq // QB
    wda = wd[:cs]
    wdb = wd[cs:]
    row = lambda i: (i, 0)
    full = lambda i: (0, 0)
    in_specs = [
        pl.BlockSpec((QB, 3), row),
        pl.BlockSpec((3, nb), full),
        pl.BlockSpec((nb, cb), full),
        pl.BlockSpec((QB, cs), row),
        pl.BlockSpec((cs, co), full),
        pl.BlockSpec((cb, co), full),
        pl.BlockSpec((1, co), full),
        pl.BlockSpec((co, co), full),
        pl.BlockSpec((1, co), full),
    ]
    args = [qpos, bpos.T, xb, xskip, wda, wdb, bd[None, :], wdr, bdr[None, :]]
    if head is not None:
        wm1, bm1, wm2, bm2, wm3, bm3 = head
        cm = wm1.shape[1]
        cl = wm3.shape[1]
        in_specs += [
            pl.BlockSpec((co, cm), full), pl.BlockSpec((1, cm), full),
            pl.BlockSpec((cm, cm), full), pl.BlockSpec((1, cm), full),
            pl.BlockSpec((cm, cl), full), pl.BlockSpec((1, cl), full),
        ]
        args += [wm1, bm1[None, :], wm2, bm2[None, :], wm3, bm3[None, :]]
        out_c = cl
    else:
        out_c = co
    body = functools.partial(_dec_body, with_head=head is not None)
    return pl.pallas_call(
        body,
        grid=(grid,),
        in_specs=in_specs,
        out_specs=pl.BlockSpec((QB, out_c), row),
        out_shape=jax.ShapeDtypeStruct((nq, out_c), _F32),
    )(*args)


def kernel(pos, rgb, normals, batch, ptr, params):
    p = params
    x, y0 = _encode(rgb, normals, pos, p)

    pos1 = pos.reshape(N1, 4, 3)[:, 0, :]
    pos2 = pos1.reshape(N2, 4, 3)[:, 0, :]

    idx0 = _knn_topk(pos1, pos, K_NBR)
    g0 = _sc_gather(y0, idx0.reshape(-1))
    g0 = g0.reshape(K_NBR, N1, 2 * HID)
    xpool0 = x.reshape(N1, 4, HID)[:, 0, :]
    wm1x = p['W_msg1'][:2 * HID]
    wm1p = p['W_msg1'][2 * HID:]
    x1, y1 = _gapl(g0, pos1, xpool0, p['W_msg0'][HID:], p['b_msg0'],
                   p['a0'], p['W_self0'], p['b_self0'],
                   p['W_r0'], p['b_r0'], p['W_res0'], p['b_res0'],
                   wyx=wm1x, wyp=wm1p)

    idx1 = _knn_topk(pos2, pos1, K_NBR)
    g1 = _sc_gather(y1, idx1.reshape(-1))
    g1 = g1.reshape(K_NBR, N2, 2 * HID)
    xpool1 = x1.reshape(N2, 4, 2 * HID)[:, 0, :]
    (x2,) = _gapl(g1, pos2, xpool1, wm1p, p['b_msg1'],
                  p['a1'], p['W_self1'], p['b_self1'],
                  p['W_r1'], p['b_r1'], p['W_res1'], p['b_res1'])

    xo1 = _decode(pos1, pos2, x2, x1, p['W_d0'], p['b_d0'],
                  p['W_dr0'], p['b_dr0'])
    wm3 = jnp.pad(p['W_m3'], ((0, 0), (0, 128 - p['W_m3'].shape[1])))
    bm3 = jnp.pad(p['b_m3'], (0, 128 - p['b_m3'].shape[0]))
    out = _decode(pos, pos1, xo1, x, p['W_d1'], p['b_d1'],
                  p['W_dr1'], p['b_dr1'],
                  head=(p['W_m1'], p['b_m1'], p['W_m2'], p['b_m2'], wm3, bm3))
    return out[:, :p['W_m3'].shape[1]]
